# Initial kernel scaffold; baseline (speedup 1.0000x reference)
import jax
import jax.numpy as jnp
from jax.experimental import pallas as pl


def kernel(x, bn_gamma, bn_beta, edge_weights):
    raise NotImplementedError("write your pallas kernel here")


# speedup vs baseline: 2.8250x; 2.8250x over previous
import jax
import jax.numpy as jnp
import numpy as np
from itertools import combinations
from jax.experimental import pallas as pl

_NF = 26
_ED = 64
_B = 4096
_FP = 32
_Q = _NF * _FP
_BB = 512
_NB = _B // _BB

_pairs = list(combinations(range(_NF), 2))
_COLS_NP = np.array([p[0] for p in _pairs], dtype=np.int32)
_ROWS_NP = np.array([p[1] for p in _pairs], dtype=np.int32)


def _moments_kernel(xt_ref, g_ref, s1_ref, s2_ref):
    xb = xt_ref[...]
    zpad = jnp.zeros((_FP - _NF, _BB), jnp.float32)
    gs = []
    for j in range(_NF):
        xj = xb[j]
        g = jnp.sum(xb * xj[None, :, :], axis=1)
        gs.append(jnp.concatenate([g, zpad], axis=0))
    G = jnp.concatenate(gs, axis=0)
    g_ref[...] = G
    ps1 = jnp.broadcast_to(jnp.sum(G, axis=1, keepdims=True), (_Q, 128))
    ps2 = jnp.broadcast_to(jnp.sum(G * G, axis=1, keepdims=True), (_Q, 128))

    @pl.when(pl.program_id(0) == 0)
    def _init():
        s1_ref[...] = ps1
        s2_ref[...] = ps2

    @pl.when(pl.program_id(0) != 0)
    def _acc():
        s1_ref[...] += ps1
        s2_ref[...] += ps2


def _output_kernel(g_ref, s1_ref, s2_ref, wm_ref, gm_ref, bm_ref, out_ref):
    s1 = s1_ref[:, 0:1]
    s2 = s2_ref[:, 0:1]
    mean = s1 * (1.0 / _B)
    var = s2 * (1.0 / _B) - mean * mean
    rstd = jax.lax.rsqrt(var + 1e-5)
    wm = wm_ref[:, 0:1]
    gm = gm_ref[:, 0:1]
    bm = bm_ref[:, 0:1]
    alpha = wm * gm * rstd
    cval = jnp.sum(wm * (bm - gm * mean * rstd))
    G = g_ref[...]
    contrib = jnp.sum(G * alpha, axis=0) + cval
    out_ref[...] = contrib.reshape(1, _BB)


def kernel(x, bn_gamma, bn_beta, edge_weights):
    xt = jnp.transpose(x, (1, 2, 0))

    def scat(v):
        m = jnp.zeros((_NF, _FP), jnp.float32)
        m = m.at[_ROWS_NP, _COLS_NP].set(v)
        return jnp.broadcast_to(m.reshape(_Q, 1), (_Q, 128))

    wm = scat(edge_weights)
    gm = scat(bn_gamma)
    bm = scat(bn_beta)

    g, s1, s2 = pl.pallas_call(
        _moments_kernel,
        grid=(_NB,),
        in_specs=[pl.BlockSpec((_NF, _ED, _BB), lambda i: (0, 0, i))],
        out_specs=[
            pl.BlockSpec((_Q, _BB), lambda i: (0, i)),
            pl.BlockSpec((_Q, 128), lambda i: (0, 0)),
            pl.BlockSpec((_Q, 128), lambda i: (0, 0)),
        ],
        out_shape=[
            jax.ShapeDtypeStruct((_Q, _B), jnp.float32),
            jax.ShapeDtypeStruct((_Q, 128), jnp.float32),
            jax.ShapeDtypeStruct((_Q, 128), jnp.float32),
        ],
    )(xt)

    out = pl.pallas_call(
        _output_kernel,
        grid=(_NB,),
        in_specs=[
            pl.BlockSpec((_Q, _BB), lambda i: (0, i)),
            pl.BlockSpec((_Q, 128), lambda i: (0, 0)),
            pl.BlockSpec((_Q, 128), lambda i: (0, 0)),
            pl.BlockSpec((_Q, 128), lambda i: (0, 0)),
            pl.BlockSpec((_Q, 128), lambda i: (0, 0)),
            pl.BlockSpec((_Q, 128), lambda i: (0, 0)),
        ],
        out_specs=pl.BlockSpec((1, _BB), lambda i: (0, i)),
        out_shape=jax.ShapeDtypeStruct((1, _B), jnp.float32),
    )(g, s1, s2, wm, gm, bm)

    return out.reshape(_B, 1)


# speedup vs baseline: 4.7123x; 1.6681x over previous
import jax
import jax.numpy as jnp
import numpy as np
from itertools import combinations
from jax.experimental import pallas as pl

_NF = 26
_ED = 64
_B = 4096
_BB = 512
_NB = _B // _BB

_H = [0] + [((j + 7) // 8) * 8 for j in range(1, _NF)]
_OFF = np.concatenate([[0], np.cumsum(_H)]).astype(np.int32)
_QP = int(_OFF[_NF])

_pairs = list(combinations(range(_NF), 2))
_COLS_NP = np.array([p[0] for p in _pairs], dtype=np.int32)
_ROWS_NP = np.array([p[1] for p in _pairs], dtype=np.int32)
_QIDX_NP = _OFF[_ROWS_NP] + _COLS_NP


def _moments_kernel(x_ref, g_ref, s1_ref, s2_ref):
    xb = x_ref[...]
    xt = jnp.transpose(xb)
    x3 = xt.reshape(_NF, _ED, _BB)
    gs = []
    for j in range(1, _NF):
        xj = x3[j]
        g = jnp.sum(x3[:j] * xj[None, :, :], axis=1)
        pad = _H[j] - j
        if pad:
            g = jnp.concatenate([g, jnp.zeros((pad, _BB), jnp.float32)], axis=0)
        gs.append(g)
    G = jnp.concatenate(gs, axis=0)
    g_ref[...] = G
    ps1 = jnp.broadcast_to(jnp.sum(G, axis=1, keepdims=True), (_QP, 128))
    ps2 = jnp.broadcast_to(jnp.sum(G * G, axis=1, keepdims=True), (_QP, 128))

    @pl.when(pl.program_id(0) == 0)
    def _init():
        s1_ref[...] = ps1
        s2_ref[...] = ps2

    @pl.when(pl.program_id(0) != 0)
    def _acc():
        s1_ref[...] += ps1
        s2_ref[...] += ps2


def _output_kernel(g_ref, s1_ref, s2_ref, wm_ref, gm_ref, bm_ref, out_ref):
    s1 = s1_ref[:, 0:1]
    s2 = s2_ref[:, 0:1]
    mean = s1 * (1.0 / _B)
    var = s2 * (1.0 / _B) - mean * mean
    rstd = jax.lax.rsqrt(var + 1e-5)
    wm = wm_ref[:, 0:1]
    gm = gm_ref[:, 0:1]
    bm = bm_ref[:, 0:1]
    alpha = wm * gm * rstd
    cval = jnp.sum(wm * (bm - gm * mean * rstd))
    G = g_ref[...]
    contrib = jnp.sum(G * alpha, axis=0) + cval
    out_ref[...] = contrib.reshape(1, _BB)


def kernel(x, bn_gamma, bn_beta, edge_weights):
    xf = x.reshape(_B, _NF * _ED)

    def scat(v):
        m = jnp.zeros((_QP,), jnp.float32).at[_QIDX_NP].set(v)
        return jnp.broadcast_to(m.reshape(_QP, 1), (_QP, 128))

    wm = scat(edge_weights)
    gm = scat(bn_gamma)
    bm = scat(bn_beta)

    g, s1, s2 = pl.pallas_call(
        _moments_kernel,
        grid=(_NB,),
        in_specs=[pl.BlockSpec((_BB, _NF * _ED), lambda i: (i, 0))],
        out_specs=[
            pl.BlockSpec((_QP, _BB), lambda i: (0, i)),
            pl.BlockSpec((_QP, 128), lambda i: (0, 0)),
            pl.BlockSpec((_QP, 128), lambda i: (0, 0)),
        ],
        out_shape=[
            jax.ShapeDtypeStruct((_QP, _B), jnp.float32),
            jax.ShapeDtypeStruct((_QP, 128), jnp.float32),
            jax.ShapeDtypeStruct((_QP, 128), jnp.float32),
        ],
    )(xf)

    out = pl.pallas_call(
        _output_kernel,
        grid=(_NB,),
        in_specs=[
            pl.BlockSpec((_QP, _BB), lambda i: (0, i)),
            pl.BlockSpec((_QP, 128), lambda i: (0, 0)),
            pl.BlockSpec((_QP, 128), lambda i: (0, 0)),
            pl.BlockSpec((_QP, 128), lambda i: (0, 0)),
            pl.BlockSpec((_QP, 128), lambda i: (0, 0)),
            pl.BlockSpec((_QP, 128), lambda i: (0, 0)),
        ],
        out_specs=pl.BlockSpec((1, _BB), lambda i: (0, i)),
        out_shape=jax.ShapeDtypeStruct((1, _B), jnp.float32),
    )(g, s1, s2, wm, gm, bm)

    return out.reshape(_B, 1)
